# Initial kernel scaffold; baseline (speedup 1.0000x reference)
import jax
import jax.numpy as jnp
from jax.experimental import pallas as pl


def kernel(tokens_list, table):
    raise NotImplementedError("write your pallas kernel here")


# speedup vs baseline: 5.3907x; 5.3907x over previous
import functools

import jax
import jax.numpy as jnp
from jax import lax
from jax.experimental import pallas as pl
from jax.experimental.pallas import tpu as pltpu
from jax.experimental.pallas import tpu_sc as plsc

BATCH = 4096
HIST = 50
DIM = 64
LANES = 16
NC = 2
NS = 16
NW = NC * NS
PER_W = BATCH // NW
DREG = DIM // LANES


def _emb_body(tokens_hbm, table_hbm, out_hbm, idx_v, rows_v, out_v, sem):
    wid = lax.axis_index("s") * NC + lax.axis_index("c")
    base = wid * PER_W
    pltpu.sync_copy(tokens_hbm.at[pl.ds(base, PER_W)], idx_v)

    def elem(e, carry):
        pltpu.async_copy(table_hbm.at[idx_v.at[e]], rows_v, sem).wait()

        def tok(t, accs):
            return tuple(accs[d] + rows_v[t, pl.ds(d * LANES, LANES)]
                         for d in range(DREG))

        accs = lax.fori_loop(
            0, HIST, tok,
            tuple(jnp.zeros((LANES,), jnp.float32) for _ in range(DREG)))
        for d in range(DREG):
            out_v[e, pl.ds(d * LANES, LANES)] = accs[d] * (1.0 / HIST)
        return carry

    lax.fori_loop(0, PER_W, elem, 0)
    pltpu.sync_copy(out_v, out_hbm.at[pl.ds(base, PER_W)])


@functools.partial(
    pl.kernel,
    out_type=jax.ShapeDtypeStruct((BATCH, DIM), jnp.float32),
    mesh=plsc.VectorSubcoreMesh(core_axis_name="c", subcore_axis_name="s"),
    scratch_types=[
        pltpu.VMEM((PER_W, HIST), jnp.int32),
        pltpu.VMEM((HIST, DIM), jnp.float32),
        pltpu.VMEM((PER_W, DIM), jnp.float32),
        pltpu.SemaphoreType.DMA,
    ],
    compiler_params=pltpu.CompilerParams(use_tc_tiling_on_sc=False),
)
def _emb(tokens_hbm, table_hbm, out_hbm, idx_v, rows_v, out_v, sem):
    _emb_body(tokens_hbm, table_hbm, out_hbm, idx_v, rows_v, out_v, sem)


def kernel(tokens_list, table):
    return _emb(tokens_list, table)


# speedup vs baseline: 10.3136x; 1.9132x over previous
import functools

import jax
import jax.numpy as jnp
from jax import lax
from jax.experimental import pallas as pl
from jax.experimental.pallas import tpu as pltpu
from jax.experimental.pallas import tpu_sc as plsc

BATCH = 4096
HIST = 50
DIM = 64
LANES = 16
NC = 2
NS = 16
NW = NC * NS
PER_W = BATCH // NW
DREG = DIM // LANES
INV_HIST = 1.0 / HIST


def _emb_body(tokens_hbm, table_hbm, out_hbm, idx_v, acc_v, sem):
    wid = lax.axis_index("s") * NC + lax.axis_index("c")
    base = wid * PER_W
    pltpu.sync_copy(tokens_hbm.at[:, pl.ds(base, PER_W)], idx_v)

    zeros = jnp.zeros((LANES,), jnp.float32)

    def zero_elem(e, carry):
        for d in range(DREG):
            acc_v[e, pl.ds(d * LANES, LANES)] = zeros
        return carry

    lax.fori_loop(0, PER_W, zero_elem, 0)

    def fire(t, carry):
        pltpu.async_copy(table_hbm.at[idx_v.at[t]], acc_v, sem, add=True)
        return carry

    lax.fori_loop(0, HIST, fire, 0)

    def drain(t, carry):
        pltpu.make_async_copy(table_hbm.at[idx_v.at[t]], acc_v, sem).wait()
        return carry

    lax.fori_loop(0, HIST, drain, 0)

    def scale(e, carry):
        for d in range(DREG):
            sl = pl.ds(d * LANES, LANES)
            acc_v[e, sl] = acc_v[e, sl] * INV_HIST
        return carry

    lax.fori_loop(0, PER_W, scale, 0)
    pltpu.sync_copy(acc_v, out_hbm.at[pl.ds(base, PER_W)])


@functools.partial(
    pl.kernel,
    out_type=jax.ShapeDtypeStruct((BATCH, DIM), jnp.float32),
    mesh=plsc.VectorSubcoreMesh(core_axis_name="c", subcore_axis_name="s"),
    scratch_types=[
        pltpu.VMEM((HIST, PER_W), jnp.int32),
        pltpu.VMEM((PER_W, DIM), jnp.float32),
        pltpu.SemaphoreType.DMA,
    ],
    compiler_params=pltpu.CompilerParams(use_tc_tiling_on_sc=False),
)
def _emb(tokens_hbm, table_hbm, out_hbm, idx_v, acc_v, sem):
    _emb_body(tokens_hbm, table_hbm, out_hbm, idx_v, acc_v, sem)


def kernel(tokens_list, table):
    return _emb(tokens_list.T, table)
